# Initial kernel scaffold; baseline (speedup 1.0000x reference)
import jax
import jax.numpy as jnp
from jax.experimental import pallas as pl


def kernel(window, enc0_Wsrc, enc0_Wdst, enc0_asrc, enc0_adst, enc1_Wsrc, enc1_Wdst, enc1_asrc, enc1_adst, qkv_W, qkv_b, proj_W, proj_b, ffn1_W, ffn1_b, ffn2_W, ffn2_b, norm1_g, norm1_b, norm2_g, norm2_b, dec0_Wsrc, dec0_Wdst, dec0_asrc, dec0_adst, dec1_Wsrc, dec1_Wdst, dec1_asrc, dec1_adst, cls_W, cls_b):
    raise NotImplementedError("write your pallas kernel here")


# speedup vs baseline: 26.1387x; 26.1387x over previous
import jax
import jax.numpy as jnp
from jax.experimental import pallas as pl

_T = 2048
_HID = 256
_OUT_CH = 128
_NHEAD = 2
_HD = _HID // _NHEAD
_QBLK = 256
_NEG = -1e30


def _shift_up(a, fill=0.0):
    pad = jnp.full((1, a.shape[1]), fill, a.dtype)
    return jnp.concatenate([a[1:], pad], axis=0)


def _shift_dn(a, fill=0.0):
    pad = jnp.full((1, a.shape[1]), fill, a.dtype)
    return jnp.concatenate([pad, a[:-1]], axis=0)


def _gat(x, WsT, WdT, a_src_flat, a_dst_flat, heads, ch, relu):
    xs = jnp.dot(x, WsT, preferred_element_type=jnp.float32)
    xd = jnp.dot(x, WdT, preferred_element_type=jnp.float32)
    ps = xs * a_src_flat
    pd = xd * a_dst_flat
    acc = None
    for h in range(heads):
        sl = slice(h * ch, (h + 1) * ch)
        ss = jnp.sum(ps[:, sl], axis=1, keepdims=True)
        sd = jnp.sum(pd[:, sl], axis=1, keepdims=True)
        xsh = xs[:, sl]
        l1 = ss + _shift_up(sd, _NEG)
        l2 = _shift_up(ss, _NEG) + sd
        l3 = ss + sd
        m = jnp.maximum(jnp.max(l3), jnp.maximum(jnp.max(l1), jnp.max(l2)))
        e1 = jnp.exp(l1 - m)
        e2 = jnp.exp(l2 - m)
        e3 = jnp.exp(l3 - m)
        z = jnp.sum(e1) + jnp.sum(e2) + jnp.sum(e3)
        out = e3 * xsh + _shift_dn(e1 * xsh) + e2 * _shift_up(xsh)
        out = out * (1.0 / z)
        acc = out if acc is None else acc + out
    if heads > 1:
        acc = acc * (1.0 / heads)
    if relu:
        acc = jnp.maximum(acc, 0.0)
    return acc


def _layernorm(x, g, b):
    m = jnp.mean(x, axis=1, keepdims=True)
    c = x - m
    v = jnp.mean(c * c, axis=1, keepdims=True)
    return c * jax.lax.rsqrt(v + 1e-5) * g + b


def _body(window_ref,
          e0WsT, e0WdT, e0as, e0ad,
          e1WsT, e1WdT, e1as, e1ad,
          qkvWT, qkvb, projWT, projb,
          f1WT, f1b, f2WT, f2b,
          n1g, n1b, n2g, n2b,
          d0WsT, d0WdT, d0as, d0ad,
          d1WsT, d1WdT, d1as, d1ad,
          clsWT, clsb,
          out_ref, logits_ref):
    x = _gat(window_ref[...], e0WsT[...], e0WdT[...], e0as[...], e0ad[...],
             _NHEAD, _HID, relu=True)
    x = _gat(x, e1WsT[...], e1WdT[...], e1as[...], e1ad[...],
             _NHEAD, _HID, relu=True)

    res = x
    xn = _layernorm(x, n1g[...], n1b[...])
    qkv = jnp.dot(xn, qkvWT[...], preferred_element_type=jnp.float32) + qkvb[...]
    scale = 1.0 / (_HD ** 0.5)
    head_outs = []
    for h in range(_NHEAD):
        qh = qkv[:, h * _HD:(h + 1) * _HD] * scale
        kh = qkv[:, _HID + h * _HD:_HID + (h + 1) * _HD]
        vh = qkv[:, 2 * _HID + h * _HD:2 * _HID + (h + 1) * _HD]
        blocks = []
        for b in range(_T // _QBLK):
            qb = qh[b * _QBLK:(b + 1) * _QBLK]
            s = jax.lax.dot_general(qb, kh, (((1,), (1,)), ((), ())),
                                    preferred_element_type=jnp.float32)
            mx = jnp.max(s, axis=1, keepdims=True)
            e = jnp.exp(s - mx)
            p = e * (1.0 / jnp.sum(e, axis=1, keepdims=True))
            blocks.append(jnp.dot(p, vh, preferred_element_type=jnp.float32))
        head_outs.append(jnp.concatenate(blocks, axis=0))
    o = jnp.concatenate(head_outs, axis=1)
    x = res + jnp.dot(o, projWT[...], preferred_element_type=jnp.float32) + projb[...]
    res = x
    xn = _layernorm(x, n2g[...], n2b[...])
    f = jnp.dot(xn, f1WT[...], preferred_element_type=jnp.float32) + f1b[...]
    f = 0.5 * f * (1.0 + jax.lax.erf(f * (2.0 ** -0.5)))
    x = res + jnp.dot(f, f2WT[...], preferred_element_type=jnp.float32) + f2b[...]

    h_cls = jnp.mean(x, axis=0, keepdims=True)
    logits_ref[...] = (jnp.dot(h_cls, clsWT[...],
                               preferred_element_type=jnp.float32) + clsb[...])

    x = _gat(x, d0WsT[...], d0WdT[...], d0as[...], d0ad[...],
             _NHEAD, _HID, relu=True)
    x = _gat(x, d1WsT[...], d1WdT[...], d1as[...], d1ad[...],
             1, _OUT_CH, relu=False)
    out_ref[...] = x


def kernel(window, enc0_Wsrc, enc0_Wdst, enc0_asrc, enc0_adst,
           enc1_Wsrc, enc1_Wdst, enc1_asrc, enc1_adst,
           qkv_W, qkv_b, proj_W, proj_b, ffn1_W, ffn1_b, ffn2_W, ffn2_b,
           norm1_g, norm1_b, norm2_g, norm2_b,
           dec0_Wsrc, dec0_Wdst, dec0_asrc, dec0_adst,
           dec1_Wsrc, dec1_Wdst, dec1_asrc, dec1_adst, cls_W, cls_b):
    f32 = jnp.float32
    cls_WT = jnp.zeros((_HID, 128), f32).at[:, :2].set(cls_W.T)
    cls_bp = jnp.zeros((1, 128), f32).at[:, :2].set(cls_b)
    operands = (
        window,
        enc0_Wsrc.T, enc0_Wdst.T,
        enc0_asrc.reshape(1, -1), enc0_adst.reshape(1, -1),
        enc1_Wsrc.T, enc1_Wdst.T,
        enc1_asrc.reshape(1, -1), enc1_adst.reshape(1, -1),
        qkv_W.T, qkv_b.reshape(1, -1), proj_W.T, proj_b.reshape(1, -1),
        ffn1_W.T, ffn1_b.reshape(1, -1), ffn2_W.T, ffn2_b.reshape(1, -1),
        norm1_g.reshape(1, -1), norm1_b.reshape(1, -1),
        norm2_g.reshape(1, -1), norm2_b.reshape(1, -1),
        dec0_Wsrc.T, dec0_Wdst.T,
        dec0_asrc.reshape(1, -1), dec0_adst.reshape(1, -1),
        dec1_Wsrc.T, dec1_Wdst.T,
        dec1_asrc.reshape(1, -1), dec1_adst.reshape(1, -1),
        cls_WT, cls_bp,
    )
    out_h, logits_p = pl.pallas_call(
        _body,
        out_shape=(
            jax.ShapeDtypeStruct((_T, _OUT_CH), f32),
            jax.ShapeDtypeStruct((1, 128), f32),
        ),
    )(*operands)
    return (out_h.T, logits_p[0, :2])


# speedup vs baseline: 26.6862x; 1.0209x over previous
import jax
import jax.numpy as jnp
from jax.experimental import pallas as pl

_T = 2048
_HID = 256
_OUT_CH = 128
_NHEAD = 2
_HD = _HID // _NHEAD
_QBLK = 256
_NEG = -1e30


def _shift_up(a, fill=0.0):
    pad = jnp.full((1, a.shape[1]), fill, a.dtype)
    return jnp.concatenate([a[1:], pad], axis=0)


def _shift_dn(a, fill=0.0):
    pad = jnp.full((1, a.shape[1]), fill, a.dtype)
    return jnp.concatenate([pad, a[:-1]], axis=0)


def _bdot(a, b):
    return jnp.dot(a.astype(jnp.bfloat16), b.astype(jnp.bfloat16),
                   preferred_element_type=jnp.float32)


def _gat(x, WsT, WdT, a_src_flat, a_dst_flat, heads, ch, relu):
    xb = x.astype(jnp.bfloat16)
    xs = jnp.dot(xb, WsT, preferred_element_type=jnp.float32)
    xd = jnp.dot(xb, WdT, preferred_element_type=jnp.float32)
    ps = xs * a_src_flat
    pd = xd * a_dst_flat
    acc = None
    for h in range(heads):
        sl = slice(h * ch, (h + 1) * ch)
        ss = jnp.sum(ps[:, sl], axis=1, keepdims=True)
        sd = jnp.sum(pd[:, sl], axis=1, keepdims=True)
        xsh = xs[:, sl]
        l1 = ss + _shift_up(sd, _NEG)
        l2 = _shift_up(ss, _NEG) + sd
        l3 = ss + sd
        m = jnp.maximum(jnp.max(l3), jnp.maximum(jnp.max(l1), jnp.max(l2)))
        e1 = jnp.exp(l1 - m)
        e2 = jnp.exp(l2 - m)
        e3 = jnp.exp(l3 - m)
        z = jnp.sum(e1) + jnp.sum(e2) + jnp.sum(e3)
        out = e3 * xsh + _shift_dn(e1 * xsh) + e2 * _shift_up(xsh)
        out = out * (1.0 / z)
        acc = out if acc is None else acc + out
    if heads > 1:
        acc = acc * (1.0 / heads)
    if relu:
        acc = jnp.maximum(acc, 0.0)
    return acc


def _layernorm(x, g, b):
    m = jnp.mean(x, axis=1, keepdims=True)
    c = x - m
    v = jnp.mean(c * c, axis=1, keepdims=True)
    return c * jax.lax.rsqrt(v + 1e-5) * g + b


def _body(window_ref,
          e0WsT, e0WdT, e0as, e0ad,
          e1WsT, e1WdT, e1as, e1ad,
          qkvWT, qkvb, projWT, projb,
          f1WT, f1b, f2WT, f2b,
          n1g, n1b, n2g, n2b,
          d0WsT, d0WdT, d0as, d0ad,
          d1WsT, d1WdT, d1as, d1ad,
          clsWT, clsb,
          out_ref, logits_ref):
    x = _gat(window_ref[...], e0WsT[...], e0WdT[...], e0as[...], e0ad[...],
             _NHEAD, _HID, relu=True)
    x = _gat(x, e1WsT[...], e1WdT[...], e1as[...], e1ad[...],
             _NHEAD, _HID, relu=True)

    res = x
    xn = _layernorm(x, n1g[...], n1b[...])
    qkv = _bdot(xn, qkvWT[...]) + qkvb[...]
    scale = 1.0 / (_HD ** 0.5)
    head_outs = []
    for h in range(_NHEAD):
        qh = (qkv[:, h * _HD:(h + 1) * _HD] * scale).astype(jnp.bfloat16)
        kh = qkv[:, _HID + h * _HD:_HID + (h + 1) * _HD].astype(jnp.bfloat16)
        vh = qkv[:, 2 * _HID + h * _HD:2 * _HID + (h + 1) * _HD].astype(jnp.bfloat16)
        blocks = []
        for b in range(_T // _QBLK):
            qb = qh[b * _QBLK:(b + 1) * _QBLK]
            s = jax.lax.dot_general(qb, kh, (((1,), (1,)), ((), ())),
                                    preferred_element_type=jnp.float32)
            mx = jnp.max(s, axis=1, keepdims=True)
            e = jnp.exp(s - mx)
            p = e * (1.0 / jnp.sum(e, axis=1, keepdims=True))
            blocks.append(jnp.dot(p.astype(jnp.bfloat16), vh,
                                  preferred_element_type=jnp.float32))
        head_outs.append(jnp.concatenate(blocks, axis=0))
    o = jnp.concatenate(head_outs, axis=1)
    x = res + _bdot(o, projWT[...]) + projb[...]
    res = x
    xn = _layernorm(x, n2g[...], n2b[...])
    f = _bdot(xn, f1WT[...]) + f1b[...]
    f = 0.5 * f * (1.0 + jax.lax.erf(f * (2.0 ** -0.5)))
    x = res + _bdot(f, f2WT[...]) + f2b[...]

    h_cls = jnp.mean(x, axis=0, keepdims=True)
    logits_ref[...] = (jnp.dot(h_cls, clsWT[...],
                               preferred_element_type=jnp.float32) + clsb[...])

    x = _gat(x, d0WsT[...], d0WdT[...], d0as[...], d0ad[...],
             _NHEAD, _HID, relu=True)
    x = _gat(x, d1WsT[...], d1WdT[...], d1as[...], d1ad[...],
             1, _OUT_CH, relu=False)
    out_ref[...] = x


def kernel(window, enc0_Wsrc, enc0_Wdst, enc0_asrc, enc0_adst,
           enc1_Wsrc, enc1_Wdst, enc1_asrc, enc1_adst,
           qkv_W, qkv_b, proj_W, proj_b, ffn1_W, ffn1_b, ffn2_W, ffn2_b,
           norm1_g, norm1_b, norm2_g, norm2_b,
           dec0_Wsrc, dec0_Wdst, dec0_asrc, dec0_adst,
           dec1_Wsrc, dec1_Wdst, dec1_asrc, dec1_adst, cls_W, cls_b):
    f32 = jnp.float32
    bf16 = jnp.bfloat16
    cls_WT = jnp.zeros((_HID, 128), f32).at[:, :2].set(cls_W.T)
    cls_bp = jnp.zeros((1, 128), f32).at[:, :2].set(cls_b)
    operands = (
        window,
        enc0_Wsrc.T.astype(bf16), enc0_Wdst.T.astype(bf16),
        enc0_asrc.reshape(1, -1), enc0_adst.reshape(1, -1),
        enc1_Wsrc.T.astype(bf16), enc1_Wdst.T.astype(bf16),
        enc1_asrc.reshape(1, -1), enc1_adst.reshape(1, -1),
        qkv_W.T.astype(bf16), qkv_b.reshape(1, -1),
        proj_W.T.astype(bf16), proj_b.reshape(1, -1),
        ffn1_W.T.astype(bf16), ffn1_b.reshape(1, -1),
        ffn2_W.T.astype(bf16), ffn2_b.reshape(1, -1),
        norm1_g.reshape(1, -1), norm1_b.reshape(1, -1),
        norm2_g.reshape(1, -1), norm2_b.reshape(1, -1),
        dec0_Wsrc.T.astype(bf16), dec0_Wdst.T.astype(bf16),
        dec0_asrc.reshape(1, -1), dec0_adst.reshape(1, -1),
        dec1_Wsrc.T.astype(bf16), dec1_Wdst.T.astype(bf16),
        dec1_asrc.reshape(1, -1), dec1_adst.reshape(1, -1),
        cls_WT, cls_bp,
    )
    out_h, logits_p = pl.pallas_call(
        _body,
        out_shape=(
            jax.ShapeDtypeStruct((_T, _OUT_CH), f32),
            jax.ShapeDtypeStruct((1, 128), f32),
        ),
    )(*operands)
    return (out_h.T, logits_p[0, :2])


# speedup vs baseline: 33.1385x; 1.2418x over previous
import jax
import jax.numpy as jnp
from jax.experimental import pallas as pl

_T = 2048
_HID = 256
_OUT_CH = 128
_NHEAD = 2
_HD = _HID // _NHEAD
_QBLK = 256
_NEG = -1e30

_DNT = (((1,), (1,)), ((), ()))


def _bdott(a, b):
    return jax.lax.dot_general(a.astype(jnp.bfloat16), b.astype(jnp.bfloat16),
                               _DNT, preferred_element_type=jnp.float32)


def _shift_up(a, fill=0.0):
    pad = jnp.full((1, a.shape[1]), fill, a.dtype)
    return jnp.concatenate([a[1:], pad], axis=0)


def _shift_dn(a, fill=0.0):
    pad = jnp.full((1, a.shape[1]), fill, a.dtype)
    return jnp.concatenate([pad, a[:-1]], axis=0)


def _gat(x, Ws, Wd, a_src_flat, a_dst_flat, heads, ch, relu):
    xs = _bdott(x, Ws)
    xd = _bdott(x, Wd)
    ps = xs * a_src_flat
    pd = xd * a_dst_flat
    acc = None
    for h in range(heads):
        sl = slice(h * ch, (h + 1) * ch)
        ss = jnp.sum(ps[:, sl], axis=1, keepdims=True)
        sd = jnp.sum(pd[:, sl], axis=1, keepdims=True)
        xsh = xs[:, sl]
        l1 = ss + _shift_up(sd, _NEG)
        l2 = _shift_up(ss, _NEG) + sd
        l3 = ss + sd
        m = jnp.maximum(jnp.max(l3), jnp.maximum(jnp.max(l1), jnp.max(l2)))
        e1 = jnp.exp(l1 - m)
        e2 = jnp.exp(l2 - m)
        e3 = jnp.exp(l3 - m)
        z = jnp.sum(e1) + jnp.sum(e2) + jnp.sum(e3)
        out = e3 * xsh + _shift_dn(e1 * xsh) + e2 * _shift_up(xsh)
        out = out * (1.0 / z)
        acc = out if acc is None else acc + out
    if heads > 1:
        acc = acc * (1.0 / heads)
    if relu:
        acc = jnp.maximum(acc, 0.0)
    return acc


def _layernorm(x, g, b):
    m = jnp.mean(x, axis=1, keepdims=True)
    c = x - m
    v = jnp.mean(c * c, axis=1, keepdims=True)
    return c * jax.lax.rsqrt(v + 1e-5) * g + b


def _body(window_ref,
          e0Ws, e0Wd, e0as, e0ad,
          e1Ws, e1Wd, e1as, e1ad,
          qkvW, qkvb, projW, projb,
          f1W, f1b, f2W, f2b,
          n1g, n1b, n2g, n2b,
          d0Ws, d0Wd, d0as, d0ad,
          d1Ws, d1Wd, d1as, d1ad,
          clsW, clsb,
          out_ref, logits_ref):
    x = _gat(window_ref[...], e0Ws[...], e0Wd[...], e0as[...], e0ad[...],
             _NHEAD, _HID, relu=True)
    x = _gat(x, e1Ws[...], e1Wd[...], e1as[...], e1ad[...],
             _NHEAD, _HID, relu=True)

    res = x
    xn = _layernorm(x, n1g[...], n1b[...])
    qkv = _bdott(xn, qkvW[...]) + qkvb[...]
    scale = 1.0 / (_HD ** 0.5)
    head_outs = []
    for h in range(_NHEAD):
        qh = (qkv[:, h * _HD:(h + 1) * _HD] * scale).astype(jnp.bfloat16)
        kh = qkv[:, _HID + h * _HD:_HID + (h + 1) * _HD].astype(jnp.bfloat16)
        vh = qkv[:, 2 * _HID + h * _HD:2 * _HID + (h + 1) * _HD].astype(jnp.bfloat16)
        blocks = []
        for b in range(_T // _QBLK):
            qb = qh[b * _QBLK:(b + 1) * _QBLK]
            s = jax.lax.dot_general(qb, kh, _DNT,
                                    preferred_element_type=jnp.float32)
            mx = jnp.max(s, axis=1, keepdims=True)
            e = jnp.exp(s - mx)
            p = e * (1.0 / jnp.sum(e, axis=1, keepdims=True))
            blocks.append(jnp.dot(p.astype(jnp.bfloat16), vh,
                                  preferred_element_type=jnp.float32))
        head_outs.append(jnp.concatenate(blocks, axis=0))
    o = jnp.concatenate(head_outs, axis=1)
    x = res + _bdott(o, projW[...]) + projb[...]
    res = x
    xn = _layernorm(x, n2g[...], n2b[...])
    f = _bdott(xn, f1W[...]) + f1b[...]
    f = 0.5 * f * (1.0 + jax.lax.erf(f * (2.0 ** -0.5)))
    x = res + _bdott(f, f2W[...]) + f2b[...]

    h_cls = jnp.mean(x, axis=0, keepdims=True)
    lg = jax.lax.dot_general(h_cls, clsW[...], _DNT,
                             preferred_element_type=jnp.float32) + clsb[...]
    logits_ref[...] = jnp.concatenate(
        [lg, jnp.zeros((1, 126), jnp.float32)], axis=1)

    x = _gat(x, d0Ws[...], d0Wd[...], d0as[...], d0ad[...],
             _NHEAD, _HID, relu=True)
    x = _gat(x, d1Ws[...], d1Wd[...], d1as[...], d1ad[...],
             1, _OUT_CH, relu=False)
    out_ref[...] = x.T


def kernel(window, enc0_Wsrc, enc0_Wdst, enc0_asrc, enc0_adst,
           enc1_Wsrc, enc1_Wdst, enc1_asrc, enc1_adst,
           qkv_W, qkv_b, proj_W, proj_b, ffn1_W, ffn1_b, ffn2_W, ffn2_b,
           norm1_g, norm1_b, norm2_g, norm2_b,
           dec0_Wsrc, dec0_Wdst, dec0_asrc, dec0_adst,
           dec1_Wsrc, dec1_Wdst, dec1_asrc, dec1_adst, cls_W, cls_b):
    f32 = jnp.float32
    operands = (
        window,
        enc0_Wsrc, enc0_Wdst,
        enc0_asrc.reshape(1, -1), enc0_adst.reshape(1, -1),
        enc1_Wsrc, enc1_Wdst,
        enc1_asrc.reshape(1, -1), enc1_adst.reshape(1, -1),
        qkv_W, qkv_b.reshape(1, -1), proj_W, proj_b.reshape(1, -1),
        ffn1_W, ffn1_b.reshape(1, -1), ffn2_W, ffn2_b.reshape(1, -1),
        norm1_g.reshape(1, -1), norm1_b.reshape(1, -1),
        norm2_g.reshape(1, -1), norm2_b.reshape(1, -1),
        dec0_Wsrc, dec0_Wdst,
        dec0_asrc.reshape(1, -1), dec0_adst.reshape(1, -1),
        dec1_Wsrc, dec1_Wdst,
        dec1_asrc.reshape(1, -1), dec1_adst.reshape(1, -1),
        cls_W, cls_b.reshape(1, -1),
    )
    out_t, logits_p = pl.pallas_call(
        _body,
        out_shape=(
            jax.ShapeDtypeStruct((_OUT_CH, _T), f32),
            jax.ShapeDtypeStruct((1, 128), f32),
        ),
    )(*operands)
    return (out_t, logits_p[0, :2])


# speedup vs baseline: 44.5814x; 1.3453x over previous
import jax
import jax.numpy as jnp
from jax.experimental import pallas as pl

_T = 2048
_HID = 256
_OUT_CH = 128
_NHEAD = 2
_HD = _HID // _NHEAD
_QBLK = 256
_NEG = -1e30

_DNT = (((1,), (1,)), ((), ()))


def _bdott(a, b):
    return jax.lax.dot_general(a.astype(jnp.bfloat16), b.astype(jnp.bfloat16),
                               _DNT, preferred_element_type=jnp.float32)


def _shift_up(a, fill=0.0):
    pad = jnp.full((1, a.shape[1]), fill, a.dtype)
    return jnp.concatenate([a[1:], pad], axis=0)


def _shift_dn(a, fill=0.0):
    pad = jnp.full((1, a.shape[1]), fill, a.dtype)
    return jnp.concatenate([pad, a[:-1]], axis=0)


def _lshift_up(a, fill):
    pad = jnp.full((a.shape[0], 1), fill, a.dtype)
    return jnp.concatenate([a[:, 1:], pad], axis=1)


def _gat(x, Ws, Wd, a_src_flat, a_dst_flat, heads, ch, relu):
    xs = _bdott(x, Ws)
    xd = _bdott(x, Wd)
    rows = []
    for h in range(heads):
        sl = slice(h * ch, (h + 1) * ch)
        rows.append(jax.lax.dot_general(
            a_src_flat[:, sl], Ws[sl, :], (((1,), (0,)), ((), ())),
            preferred_element_type=jnp.float32))
        rows.append(jax.lax.dot_general(
            a_dst_flat[:, sl], Wd[sl, :], (((1,), (0,)), ((), ())),
            preferred_element_type=jnp.float32))
    wt = jnp.concatenate(rows, axis=0)
    sall = jax.lax.dot_general(wt, x, _DNT,
                               preferred_element_type=jnp.float32)
    hscale = 1.0 / heads
    acc = None
    for h in range(heads):
        ss = sall[2 * h:2 * h + 1, :]
        sd = sall[2 * h + 1:2 * h + 2, :]
        xsh = xs[:, h * ch:(h + 1) * ch]
        l1 = ss + _lshift_up(sd, _NEG)
        l2 = _lshift_up(ss, _NEG) + sd
        l3 = ss + sd
        m = jnp.maximum(jnp.max(l3), jnp.maximum(jnp.max(l1), jnp.max(l2)))
        e1 = jnp.exp(l1 - m)
        e2 = jnp.exp(l2 - m)
        e3 = jnp.exp(l3 - m)
        z = hscale / (jnp.sum(e1) + jnp.sum(e2) + jnp.sum(e3))
        e1c = (e1 * z).reshape(-1, 1)
        e2c = (e2 * z).reshape(-1, 1)
        e3c = (e3 * z).reshape(-1, 1)
        out = e3c * xsh + _shift_dn(e1c * xsh) + e2c * _shift_up(xsh)
        acc = out if acc is None else acc + out
    if relu:
        acc = jnp.maximum(acc, 0.0)
    return acc


def _layernorm(x, g, b):
    m = jnp.mean(x, axis=1, keepdims=True)
    c = x - m
    v = jnp.mean(c * c, axis=1, keepdims=True)
    return c * jax.lax.rsqrt(v + 1e-5) * g + b


def _body(window_ref,
          e0Ws, e0Wd, e0as, e0ad,
          e1Ws, e1Wd, e1as, e1ad,
          qkvW, qkvb, projW, projb,
          f1W, f1b, f2W, f2b,
          n1g, n1b, n2g, n2b,
          d0Ws, d0Wd, d0as, d0ad,
          d1Ws, d1Wd, d1as, d1ad,
          clsW, clsb,
          out_ref, logits_ref):
    x = _gat(window_ref[...], e0Ws[...], e0Wd[...], e0as[...], e0ad[...],
             _NHEAD, _HID, relu=True)
    x = _gat(x, e1Ws[...], e1Wd[...], e1as[...], e1ad[...],
             _NHEAD, _HID, relu=True)

    res = x
    xn = _layernorm(x, n1g[...], n1b[...])
    qkv = _bdott(xn, qkvW[...]) + qkvb[...]
    scale = 1.0 / (_HD ** 0.5)
    head_outs = []
    for h in range(_NHEAD):
        qh = (qkv[:, h * _HD:(h + 1) * _HD] * scale).astype(jnp.bfloat16)
        kh = qkv[:, _HID + h * _HD:_HID + (h + 1) * _HD].astype(jnp.bfloat16)
        vh = qkv[:, 2 * _HID + h * _HD:2 * _HID + (h + 1) * _HD].astype(jnp.bfloat16)
        blocks = []
        for b in range(_T // _QBLK):
            qb = qh[b * _QBLK:(b + 1) * _QBLK]
            s = jax.lax.dot_general(qb, kh, _DNT,
                                    preferred_element_type=jnp.float32)
            mx = jnp.max(s, axis=1, keepdims=True)
            e = jnp.exp(s - mx)
            r = 1.0 / jnp.sum(e, axis=1, keepdims=True)
            ob = jnp.dot(e.astype(jnp.bfloat16), vh,
                         preferred_element_type=jnp.float32)
            blocks.append(ob * r)
        head_outs.append(jnp.concatenate(blocks, axis=0))
    o = jnp.concatenate(head_outs, axis=1)
    x = res + _bdott(o, projW[...]) + projb[...]
    res = x
    xn = _layernorm(x, n2g[...], n2b[...])
    f = _bdott(xn, f1W[...]) + f1b[...]
    f = 0.5 * f * (1.0 + jax.lax.erf(f * (2.0 ** -0.5)))
    x = res + _bdott(f, f2W[...]) + f2b[...]

    h_cls = jnp.mean(x, axis=0, keepdims=True)
    lg = jax.lax.dot_general(h_cls, clsW[...], _DNT,
                             preferred_element_type=jnp.float32) + clsb[...]
    logits_ref[...] = jnp.concatenate(
        [lg, jnp.zeros((1, 126), jnp.float32)], axis=1)

    x = _gat(x, d0Ws[...], d0Wd[...], d0as[...], d0ad[...],
             _NHEAD, _HID, relu=True)
    x = _gat(x, d1Ws[...], d1Wd[...], d1as[...], d1ad[...],
             1, _OUT_CH, relu=False)
    out_ref[...] = x.T


def kernel(window, enc0_Wsrc, enc0_Wdst, enc0_asrc, enc0_adst,
           enc1_Wsrc, enc1_Wdst, enc1_asrc, enc1_adst,
           qkv_W, qkv_b, proj_W, proj_b, ffn1_W, ffn1_b, ffn2_W, ffn2_b,
           norm1_g, norm1_b, norm2_g, norm2_b,
           dec0_Wsrc, dec0_Wdst, dec0_asrc, dec0_adst,
           dec1_Wsrc, dec1_Wdst, dec1_asrc, dec1_adst, cls_W, cls_b):
    f32 = jnp.float32
    operands = (
        window,
        enc0_Wsrc, enc0_Wdst,
        enc0_asrc.reshape(1, -1), enc0_adst.reshape(1, -1),
        enc1_Wsrc, enc1_Wdst,
        enc1_asrc.reshape(1, -1), enc1_adst.reshape(1, -1),
        qkv_W, qkv_b.reshape(1, -1), proj_W, proj_b.reshape(1, -1),
        ffn1_W, ffn1_b.reshape(1, -1), ffn2_W, ffn2_b.reshape(1, -1),
        norm1_g.reshape(1, -1), norm1_b.reshape(1, -1),
        norm2_g.reshape(1, -1), norm2_b.reshape(1, -1),
        dec0_Wsrc, dec0_Wdst,
        dec0_asrc.reshape(1, -1), dec0_adst.reshape(1, -1),
        dec1_Wsrc, dec1_Wdst,
        dec1_asrc.reshape(1, -1), dec1_adst.reshape(1, -1),
        cls_W, cls_b.reshape(1, -1),
    )
    out_t, logits_p = pl.pallas_call(
        _body,
        out_shape=(
            jax.ShapeDtypeStruct((_OUT_CH, _T), f32),
            jax.ShapeDtypeStruct((1, 128), f32),
        ),
    )(*operands)
    return (out_t, logits_p[0, :2])
